# baseline (speedup 1.0000x reference)
import functools

import jax
import jax.numpy as jnp
from jax import lax
from jax.experimental import pallas as pl
from jax.experimental.pallas import tpu as pltpu
from jax.experimental.pallas import tpu_sc as plsc

_B = 16384
_D = 32
_EPS = 1e-5
_CHUNK = 128


def _rsqrt16(x):
    i = plsc.bitcast(x, jnp.int32)
    i = jnp.int32(0x5F3759DF) - lax.shift_right_logical(i, 1)
    y = plsc.bitcast(i, jnp.float32)
    for _ in range(3):
        y = y * (1.5 - 0.5 * x * y * y)
    return y


@functools.partial(jax.jit, static_argnums=())
def _sc_embed_ln(table, idx, b, gamma, beta):
    info = plsc.get_sparse_core_info()
    nc, ns = info.num_cores, info.num_subcores
    nw = nc * ns
    bpw = _B // nw
    nchunk = bpw // _CHUNK
    nblk = bpw // 16
    mesh = plsc.VectorSubcoreMesh(core_axis_name="c", subcore_axis_name="s")

    @functools.partial(
        pl.kernel,
        mesh=mesh,
        out_type=jax.ShapeDtypeStruct((_B, _D), jnp.float32),
        scratch_types=[
            pltpu.VMEM((nchunk, _CHUNK), jnp.int32),
            pltpu.VMEM((bpw, _D), jnp.float32),
            pltpu.VMEM((bpw, _D), jnp.float32),
            pltpu.VMEM((_D,), jnp.float32),
            pltpu.VMEM((_D,), jnp.float32),
            pltpu.VMEM((_D,), jnp.float32),
            pltpu.SemaphoreType.DMA,
        ],
        compiler_params=pltpu.CompilerParams(
            needs_layout_passes=False, use_tc_tiling_on_sc=False),
    )
    def k(table_h, idx_h, b_h, g_h, be_h, out_h,
          idx_v, rows_v, out_v, b_v, g_v, be_v, sem):
        wid = lax.axis_index("s") * nc + lax.axis_index("c")
        base = wid * bpw
        for j in range(nchunk):
            pltpu.sync_copy(idx_h.at[pl.ds(base + j * _CHUNK, _CHUNK)],
                            idx_v.at[j])
        pltpu.sync_copy(b_h, b_v)
        pltpu.sync_copy(g_h, g_v)
        pltpu.sync_copy(be_h, be_v)
        copies = [
            pltpu.async_copy(table_h.at[idx_v.at[j]],
                             rows_v.at[pl.ds(j * _CHUNK, _CHUNK)], sem)
            for j in range(nchunk)
        ]
        for c in copies:
            c.wait()

        b_lanes = [b_v[pl.ds(0, 16)], b_v[pl.ds(16, 16)]]
        g_lanes = [g_v[pl.ds(0, 16)], g_v[pl.ds(16, 16)]]
        be_lanes = [be_v[pl.ds(0, 16)], be_v[pl.ds(16, 16)]]

        def block(blk, carry):
            rid = blk * 16 + lax.iota(jnp.int32, 16)
            v = []
            s = jnp.zeros((16,), jnp.float32)
            ss = jnp.zeros((16,), jnp.float32)
            for d in range(_D):
                cid = jnp.full((16,), d, jnp.int32)
                x = plsc.load_gather(rows_v, [rid, cid]) + b_lanes[d // 16][d % 16]
                v.append(x)
                s = s + x
                ss = ss + x * x
            mean = s * (1.0 / _D)
            var = ss * (1.0 / _D) - mean * mean
            r = _rsqrt16(var + _EPS)
            for d in range(_D):
                cid = jnp.full((16,), d, jnp.int32)
                o = (v[d] - mean) * r * g_lanes[d // 16][d % 16] + be_lanes[d // 16][d % 16]
                plsc.store_scatter(out_v, [rid, cid], o)
            return carry

        lax.fori_loop(0, nblk, block, 0)
        pltpu.sync_copy(out_v, out_h.at[pl.ds(base, bpw)])

    return k(table, idx, b, gamma, beta)


def kernel(species_idx, W, b, gamma, beta):
    table = W.T
    idx = species_idx.astype(jnp.int32)
    return _sc_embed_ln(table, idx, b, gamma, beta)
